# baseline (speedup 1.0000x reference)
import functools

import jax
import jax.numpy as jnp
from jax import lax
from jax.experimental import pallas as pl
from jax.experimental.pallas import tpu as pltpu
from jax.experimental.pallas import tpu_sc as plsc

NODE_NUM = 1000000
EMB_DIM = 32
BATCH = 4096
HIST = 50

LINE = 128
RPL = LINE // EMB_DIM

NC = 2
NS = 16
NW = NC * NS
BPW = BATCH // NW

NBUF = 4


def _sc_body(table_hbm, data_hbm, out_hbm, idx_v, lines_v, offs_v, gbuf_v,
             acc_v, out_v, sems):
  wid = lax.axis_index("s") * NC + lax.axis_index("c")
  base = wid * BPW

  pltpu.sync_copy(data_hbm.at[pl.ds(base * HIST, BPW * HIST)], idx_v)

  zeros = jnp.zeros((16,), jnp.float32)

  def zbody(b, c):
    acc_v[b, pl.ds(0, 16)] = zeros
    acc_v[b, pl.ds(16, 16)] = zeros
    return c
  lax.fori_loop(0, BPW, zbody, 0, unroll=8)

  lanes50 = lax.iota(jnp.int32, 16) * HIST

  def tbody(j, c):
    for b0 in range(0, BPW, 16):
      vals = plsc.load_gather(idx_v, [lanes50 + (b0 * HIST + j)])
      lines_v[j, pl.ds(b0, 16)] = lax.shift_right_logical(vals, 2)
      offs_v[j, pl.ds(b0, 16)] = (vals & 3) * EMB_DIM
    return c
  lax.fori_loop(0, HIST, tbody, 0)

  def fire(j, slot):
    pltpu.async_copy(table_hbm.at[lines_v.at[j]], gbuf_v.at[slot],
                     sems.at[slot])

  for j in range(NBUF):
    fire(j, j)

  def loop_j(j, c):
    slot = lax.rem(j, NBUF)
    pltpu.make_async_copy(table_hbm.at[lines_v.at[j]], gbuf_v.at[slot],
                          sems.at[slot]).wait()
    for b0 in range(0, BPW, 16):
      offv = offs_v[j, pl.ds(b0, 16)]
      for i in range(16):
        b = b0 + i
        off = offv[i]
        plsc.addupdate(acc_v.at[b, pl.ds(0, 16)],
                       gbuf_v[slot, b, pl.ds(off, 16)])
        plsc.addupdate(acc_v.at[b, pl.ds(16, 16)],
                       gbuf_v[slot, b, pl.ds(off + 16, 16)])

    @pl.when(j < HIST - NBUF)
    def _():
      fire(j + NBUF, slot)
    return c

  lax.fori_loop(0, HIST, loop_j, 0)

  scale = jnp.float32(1.0 / HIST)

  def finish(b, c):
    out_v[pl.ds(b * EMB_DIM, 16)] = acc_v[b, pl.ds(0, 16)] * scale
    out_v[pl.ds(b * EMB_DIM + 16, 16)] = acc_v[b, pl.ds(16, 16)] * scale
    return c
  lax.fori_loop(0, BPW, finish, 0, unroll=4)

  pltpu.sync_copy(out_v, out_hbm.at[pl.ds(base * EMB_DIM, BPW * EMB_DIM)])


@jax.jit
def _graph_encode(data, table):
  table_lines = table.reshape(NODE_NUM // RPL, LINE)
  data_flat = data.reshape(BATCH * HIST)

  mesh = plsc.VectorSubcoreMesh(
      core_axis_name="c", subcore_axis_name="s", num_cores=NC, num_subcores=NS)
  k = pl.kernel(
      _sc_body,
      out_type=jax.ShapeDtypeStruct((BATCH * EMB_DIM,), jnp.float32),
      mesh=mesh,
      scratch_types=[
          pltpu.VMEM((BPW * HIST,), jnp.int32),
          pltpu.VMEM((HIST, BPW), jnp.int32),
          pltpu.VMEM((HIST, BPW), jnp.int32),
          pltpu.VMEM((NBUF, BPW, LINE), jnp.float32),
          pltpu.VMEM((BPW, EMB_DIM), jnp.float32),
          pltpu.VMEM((BPW * EMB_DIM,), jnp.float32),
          pltpu.SemaphoreType.DMA((NBUF,)),
      ],
      compiler_params=pltpu.CompilerParams(needs_layout_passes=False),
  )
  return k(table_lines, data_flat).reshape(BATCH, EMB_DIM)


def kernel(data, table):
  return _graph_encode(data, table)
